# baseline (speedup 1.0000x reference)
import jax
import jax.numpy as jnp
from jax.experimental import pallas as pl
from jax.experimental.pallas import tpu as pltpu

_N = 1000
_S = 32
_M = _N * _N
_B = 8192
_GRID = (_M + _B - 1) // _B
_TAIL = _M - (_GRID - 1) * _B
_G = 4
_BIG = 2**30
_MARGIN = 1e-3


def _scores(u, lg):
    uc = jnp.clip(u, 1e-06, 1.0 - 1e-06)
    return lg - jnp.log(-jnp.log(uc))


def _update(best_ref, idx_ref, lo, hi, s, i):
    m = jnp.max(s, axis=1, keepdims=True)
    col = jax.lax.broadcasted_iota(jnp.int32, s.shape, 1)
    cand = jnp.where(s == m, i * _B + col, _BIG)
    ci = jnp.min(cand, axis=1, keepdims=True)
    better = m > best_ref[lo:hi]
    best_ref[lo:hi] = jnp.where(better, m, best_ref[lo:hi])
    idx_ref[lo:hi] = jnp.where(better, ci, idx_ref[lo:hi])


def _body(u_ref, l_ref, x_ref, y_ref, best_ref, idx_ref):
    i = pl.program_id(0)

    @pl.when(i == 0)
    def _init():
        best_ref[...] = jnp.full((_S, 1), -jnp.inf, jnp.float32)
        idx_ref[...] = jnp.zeros((_S, 1), jnp.int32)

    @pl.when(i < _GRID - 1)
    def _mid():
        u = u_ref[...]
        lg = l_ref[...]
        lmax = jnp.max(lg)
        rmax = jnp.max(u, axis=1, keepdims=True)
        rmc = jnp.clip(rmax, 1e-06, 1.0 - 1e-06)
        bound = lmax - jnp.log(-jnp.log(rmc)) + _MARGIN
        need = bound > best_ref[...]

        for j in range(_S // _G):
            lo = j * _G

            @pl.when(jnp.any(need[lo:lo + _G]))
            def _score(lo=lo):
                s = _scores(u[lo:lo + _G], lg)
                _update(best_ref, idx_ref, lo, lo + _G, s, i)

    @pl.when(i == _GRID - 1)
    def _last():
        u = u_ref[...]
        lg = l_ref[...]
        col = jax.lax.broadcasted_iota(jnp.int32, (_S, _B), 1)
        s = jnp.where(col < _TAIL, _scores(u, lg), -jnp.inf)
        _update(best_ref, idx_ref, 0, _S, s, i)
        ci = idx_ref[...]
        x_ref[...] = ci // _N
        y_ref[...] = ci - (ci // _N) * _N


def kernel(uniform, logits):
    lflat = logits.reshape(1, _M)
    x2, y2 = pl.pallas_call(
        _body,
        grid=(_GRID,),
        in_specs=[
            pl.BlockSpec((_S, _B), lambda i: (0, i)),
            pl.BlockSpec((1, _B), lambda i: (0, i)),
        ],
        out_specs=[
            pl.BlockSpec((_S, 1), lambda i: (0, 0)),
            pl.BlockSpec((_S, 1), lambda i: (0, 0)),
        ],
        out_shape=[
            jax.ShapeDtypeStruct((_S, 1), jnp.int32),
            jax.ShapeDtypeStruct((_S, 1), jnp.int32),
        ],
        scratch_shapes=[
            pltpu.VMEM((_S, 1), jnp.float32),
            pltpu.VMEM((_S, 1), jnp.int32),
        ],
    )(uniform, lflat)
    return x2.reshape(_S), y2.reshape(_S)


# speedup vs baseline: 1.9913x; 1.9913x over previous
import jax
import jax.numpy as jnp
from jax.experimental import pallas as pl
from jax.experimental.pallas import tpu as pltpu

_N = 1000
_S = 32
_M = _N * _N
_B = 32768
_GRID = (_M + _B - 1) // _B
_TAIL = _M - (_GRID - 1) * _B
_BIG = 2**30


_NLN2 = float(jnp.float32(-0.6931472))


def _scores(u, lg):
    uc = jnp.clip(u, 1e-06, 1.0 - 1e-06)
    t = jnp.log2(uc) * _NLN2
    return lg + jnp.log2(t) * _NLN2


def _body(u_ref, l_ref, x_ref, y_ref, accv_ref, acci_ref):
    i = pl.program_id(0)
    u = u_ref[...]
    lg = l_ref[...]

    @pl.when(i == 0)
    def _init():
        accv_ref[...] = _scores(u, lg)
        acci_ref[...] = jnp.zeros((_S, _B), jnp.int32)

    @pl.when(jnp.logical_and(i > 0, i < _GRID - 1))
    def _mid():
        s = _scores(u, lg)
        upd = s > accv_ref[...]
        accv_ref[...] = jnp.where(upd, s, accv_ref[...])
        acci_ref[...] = jnp.where(upd, i, acci_ref[...])

    @pl.when(i == _GRID - 1)
    def _last():
        col = jax.lax.broadcasted_iota(jnp.int32, (_S, _B), 1)
        s = jnp.where(col < _TAIL, _scores(u, lg), -jnp.inf)
        upd = s > accv_ref[...]
        av = jnp.where(upd, s, accv_ref[...])
        ai = jnp.where(upd, i, acci_ref[...])
        m = jnp.max(av, axis=1, keepdims=True)
        flat = ai * _B + col
        cand = jnp.where(av == m, flat, _BIG)
        ci = jnp.min(cand, axis=1, keepdims=True)
        x_ref[...] = ci // _N
        y_ref[...] = ci - (ci // _N) * _N


def kernel(uniform, logits):
    lflat = logits.reshape(1, _M)
    x2, y2 = pl.pallas_call(
        _body,
        grid=(_GRID,),
        in_specs=[
            pl.BlockSpec((_S, _B), lambda i: (0, i)),
            pl.BlockSpec((1, _B), lambda i: (0, i)),
        ],
        out_specs=[
            pl.BlockSpec((_S, 1), lambda i: (0, 0)),
            pl.BlockSpec((_S, 1), lambda i: (0, 0)),
        ],
        out_shape=[
            jax.ShapeDtypeStruct((_S, 1), jnp.int32),
            jax.ShapeDtypeStruct((_S, 1), jnp.int32),
        ],
        scratch_shapes=[
            pltpu.VMEM((_S, _B), jnp.float32),
            pltpu.VMEM((_S, _B), jnp.int32),
        ],
    )(uniform, lflat)
    return x2.reshape(_S), y2.reshape(_S)
